# baseline (speedup 1.0000x reference)
import functools

import jax
import jax.numpy as jnp
from jax import lax
from jax.experimental import pallas as pl
from jax.experimental.pallas import tpu as pltpu
from jax.experimental.pallas import tpu_sc as plsc

NUM_USERS = 25000
NUM_ITEMS = 75000
N = NUM_USERS + NUM_ITEMS
NP = 100096
E = 3200000
D = 16

NC = 2
NS = 16
NW = NC * NS

C = 128
CPB = 4
CPW = 792
E_PAD = NW * CPW * C
BPW = CPW // CPB
THIRD = BPW // 3

ROWS_PER_SUB = NP // NS

_mesh = plsc.VectorSubcoreMesh(core_axis_name="c", subcore_axis_name="s")
_params = pltpu.CompilerParams(use_tc_tiling_on_sc=False)


@functools.partial(
    pl.kernel,
    out_type=jax.ShapeDtypeStruct((2 * NP, D), jnp.float32),
    mesh=_mesh,
    compiler_params=_params,
    scratch_types=[
        pltpu.VMEM_SHARED((NP, D), jnp.float32),
        pltpu.VMEM((3, CPB, C), jnp.int32),
        pltpu.VMEM((3, CPB, C), jnp.int32),
        pltpu.VMEM((3, CPB, C), jnp.float32),
        pltpu.VMEM((3, CPB, C, D), jnp.float32),
        pltpu.SemaphoreType.DMA,
        pltpu.SemaphoreType.DMA,
        pltpu.SemaphoreType.DMA,
        pltpu.SemaphoreType.DMA,
        pltpu.SemaphoreType.DMA,
        pltpu.SemaphoreType.DMA,
        pltpu.SemaphoreType.DMA,
        pltpu.SemaphoreType.DMA,
        pltpu.SemaphoreType.DMA,
        pltpu.SemaphoreType.DMA,
        pltpu.SemaphoreType.DMA,
        pltpu.SemaphoreType.DMA,
        pltpu.SemaphoreType.DMA,
    ],
)
def _propagate(x_hbm, col_hbm, row_hbm, val_hbm, zero_hbm, out_hbm,
               acc_sh, col_b, row_b, val_b, gbuf,
               zsem, cv0, cv1, cv2, r0, r1, r2, g0, g1, g2, s0, s1, s2):
    cv_sems, r_sems = (cv0, cv1, cv2), (r0, r1, r2)
    g_sems, s_sems = (g0, g1, g2), (s0, s1, s2)
    c = lax.axis_index("c")
    s = lax.axis_index("s")
    wid = s * NC + c

    chunk0 = wid * CPW

    def cv_issue(i, t):
        rb = chunk0 + i * CPB
        pltpu.async_copy(col_hbm.at[pl.ds(rb, CPB)], col_b.at[t], cv_sems[t])
        pltpu.async_copy(val_hbm.at[pl.ds(rb, CPB)], val_b.at[t], cv_sems[t])

    def cv_wait(i, t):
        rb = chunk0 + i * CPB
        pltpu.make_async_copy(col_hbm.at[pl.ds(rb, CPB)], col_b.at[t],
                              cv_sems[t]).wait()
        pltpu.make_async_copy(val_hbm.at[pl.ds(rb, CPB)], val_b.at[t],
                              cv_sems[t]).wait()

    def row_issue(i, t):
        rb = chunk0 + i * CPB
        pltpu.async_copy(row_hbm.at[pl.ds(rb, CPB)], row_b.at[t], r_sems[t])

    def row_wait(i, t):
        rb = chunk0 + i * CPB
        pltpu.make_async_copy(row_hbm.at[pl.ds(rb, CPB)], row_b.at[t],
                              r_sems[t]).wait()

    def g_issue(t):
        for j in range(CPB):
            pltpu.async_copy(x_hbm.at[col_b.at[t, j]], gbuf.at[t, j],
                             g_sems[t])

    def g_wait(t):
        for j in range(CPB):
            pltpu.make_async_copy(x_hbm.at[col_b.at[t, j]], gbuf.at[t, j],
                                  g_sems[t]).wait()

    def s_issue(t):
        for j in range(CPB):
            pltpu.async_copy(gbuf.at[t, j], acc_sh.at[row_b.at[t, j]],
                             s_sems[t], add=True)

    def s_wait(t):
        for j in range(CPB):
            pltpu.make_async_copy(gbuf.at[t, j], acc_sh.at[row_b.at[t, j]],
                                  s_sems[t]).wait()

    def scale(t):
        for j in range(CPB):
            for g in range(C // 16):
                vv = val_b[t, j, pl.ds(g * 16, 16)]
                for e in range(16):
                    i = g * 16 + e
                    bv = jnp.broadcast_to(vv[e], (16,))
                    gbuf[t, j, i, :] = gbuf[t, j, i, :] * bv

    zh = pltpu.async_copy(
        zero_hbm.at[pl.ds(s * ROWS_PER_SUB, ROWS_PER_SUB)],
        acc_sh.at[pl.ds(s * ROWS_PER_SUB, ROWS_PER_SUB)], zsem)
    cv_issue(0, 0)
    cv_issue(1, 1)
    cv_issue(2, 2)
    row_issue(0, 0)
    cv_wait(0, 0)
    g_issue(0)
    cv_wait(1, 1)
    g_issue(1)
    zh.wait()
    plsc.subcore_barrier()

    @pl.loop(0, THIRD)
    def _triple(k):
        for t in (0, 1, 2):
            b = 3 * k + t
            u = (t + 1) % 3
            w = (t + 2) % 3
            if t == 0:
                @pl.when(k >= 1)
                def _():
                    s_wait(w)
            else:
                s_wait(w)
            if t == 2:
                @pl.when(k < THIRD - 1)
                def _():
                    row_issue(b + 1, u)
            else:
                row_issue(b + 1, u)
            if t == 0:
                cv_wait(b + 2, w)
                g_issue(w)
            else:
                @pl.when(k < THIRD - 1)
                def _():
                    cv_wait(b + 2, w)
                    g_issue(w)
            g_wait(t)
            row_wait(b, t)
            scale(t)
            s_issue(t)

            @pl.when(k < THIRD - 1)
            def _():
                cv_issue(b + 3, t)

    s_wait(2)

    plsc.subcore_barrier()
    pltpu.sync_copy(acc_sh.at[pl.ds(s * ROWS_PER_SUB, ROWS_PER_SUB)],
                    out_hbm.at[pl.ds(c * NP + s * ROWS_PER_SUB, ROWS_PER_SUB)])


_RPW = NP // NW
_RB = 184


@functools.partial(
    pl.kernel,
    out_type=jax.ShapeDtypeStruct((NP, D), jnp.float32),
    mesh=_mesh,
    compiler_params=_params,
    scratch_types=[
        pltpu.VMEM((_RB, D), jnp.float32),
        pltpu.VMEM((_RB, D), jnp.float32),
    ],
)
def _combine(p_hbm, out_hbm, a_b, b_b):
    c = lax.axis_index("c")
    s = lax.axis_index("s")
    wid = s * NC + c
    base = wid * _RPW

    @pl.loop(0, _RPW // _RB)
    def _blk(b):
        r0 = base + b * _RB
        pltpu.sync_copy(p_hbm.at[pl.ds(r0, _RB)], a_b)
        pltpu.sync_copy(p_hbm.at[pl.ds(NP + r0, _RB)], b_b)

        @pl.loop(0, _RB)
        def _row(i):
            a_b[i, :] = a_b[i, :] + b_b[i, :]

        pltpu.sync_copy(a_b, out_hbm.at[pl.ds(r0, _RB)])


@functools.partial(
    pl.kernel,
    out_type=jax.ShapeDtypeStruct((NP, D), jnp.float32),
    mesh=_mesh,
    compiler_params=_params,
    scratch_types=[
        pltpu.VMEM((_RB, D), jnp.float32),
        pltpu.VMEM((_RB, D), jnp.float32),
        pltpu.VMEM((_RB, D), jnp.float32),
        pltpu.VMEM((_RB, D), jnp.float32),
    ],
)
def _final_mean(x0_hbm, x1_hbm, q_hbm, out_hbm, a_b, b_b, c_b, d_b):
    c = lax.axis_index("c")
    s = lax.axis_index("s")
    wid = s * NC + c
    base = wid * _RPW
    third = jnp.float32(1.0 / 3.0)

    @pl.loop(0, _RPW // _RB)
    def _blk(b):
        r0 = base + b * _RB
        pltpu.sync_copy(x0_hbm.at[pl.ds(r0, _RB)], a_b)
        pltpu.sync_copy(x1_hbm.at[pl.ds(r0, _RB)], b_b)
        pltpu.sync_copy(q_hbm.at[pl.ds(r0, _RB)], c_b)
        pltpu.sync_copy(q_hbm.at[pl.ds(NP + r0, _RB)], d_b)

        @pl.loop(0, _RB)
        def _row(i):
            acc = (a_b[i, :] + b_b[i, :]) + (c_b[i, :] + d_b[i, :])
            a_b[i, :] = acc * third

        pltpu.sync_copy(a_b, out_hbm.at[pl.ds(r0, _RB)])


def kernel(A_hat_indices, A_hat_values, user_emb, item_emb):
    x0 = jnp.concatenate(
        [user_emb, item_emb, jnp.zeros((NP - N, D), jnp.float32)], axis=0)

    row = A_hat_indices[0].astype(jnp.int32)
    col = A_hat_indices[1].astype(jnp.int32)
    val = A_hat_values.astype(jnp.float32)

    pad = E_PAD - E
    pad_idx = (jnp.arange(pad, dtype=jnp.int32) * 97) % N
    row_p = jnp.concatenate([row, pad_idx]).reshape(E_PAD // C, C)
    col_p = jnp.concatenate([col, pad_idx]).reshape(E_PAD // C, C)
    val_p = jnp.concatenate([val, jnp.zeros((pad,), jnp.float32)]
                            ).reshape(E_PAD // C, C)

    zeros = jnp.zeros((NP, D), jnp.float32)

    p = _propagate(x0, col_p, row_p, val_p, zeros)
    x1 = _combine(p)
    q = _propagate(x1, col_p, row_p, val_p, zeros)
    out = _final_mean(x0, x1, q)

    return (out[:NUM_USERS], out[NUM_USERS:N])


# speedup vs baseline: 1.1653x; 1.1653x over previous
import functools

import jax
import jax.numpy as jnp
from jax import lax
from jax.experimental import pallas as pl
from jax.experimental.pallas import tpu as pltpu
from jax.experimental.pallas import tpu_sc as plsc

NUM_USERS = 25000
NUM_ITEMS = 75000
N = NUM_USERS + NUM_ITEMS
NP = 100096
E = 3200000
D = 16

NC = 2
NS = 16
NW = NC * NS

C = 128
CPB = 4
CPW = 784
E_PAD = NW * CPW * C
BPW = CPW // CPB
HALF = BPW // 2

ROWS_PER_SUB = NP // NS

_mesh = plsc.VectorSubcoreMesh(core_axis_name="c", subcore_axis_name="s")
_params = pltpu.CompilerParams(use_tc_tiling_on_sc=False)


ZROWS = 512


@functools.partial(
    pl.kernel,
    out_type=jax.ShapeDtypeStruct((2 * NP, D), jnp.float32),
    mesh=_mesh,
    compiler_params=_params,
    scratch_types=[
        pltpu.VMEM_SHARED((NP, D), jnp.float32),
        pltpu.VMEM((2, CPB * C), jnp.int32),
        pltpu.VMEM((2, CPB, C), jnp.int32),
        pltpu.VMEM((2, CPB * C), jnp.float32),
        pltpu.VMEM((2, CPB, C, D), jnp.float32),
        pltpu.VMEM((ZROWS, D), jnp.float32),
        pltpu.SemaphoreType.DMA,
        pltpu.SemaphoreType.DMA,
        pltpu.SemaphoreType.DMA,
        pltpu.SemaphoreType.DMA,
        pltpu.SemaphoreType.DMA,
        pltpu.SemaphoreType.DMA,
        pltpu.SemaphoreType.DMA,
        pltpu.SemaphoreType.DMA,
        pltpu.SemaphoreType.DMA,
    ],
)
def _propagate(x_hbm, col_hbm, row_hbm, val_hbm, out_hbm,
               acc_sh, col_b, row_b, val_b, gbuf, zbuf,
               zsem, cv0, cv1, r0, r1, g0, g1, s0, s1):
    cv_sems, r_sems = (cv0, cv1), (r0, r1)
    g_sems, s_sems = (g0, g1), (s0, s1)
    c = lax.axis_index("c")
    s = lax.axis_index("s")
    wid = s * NC + c

    edge0 = wid * CPW * C

    def cv_issue(i, t):
        eb = edge0 + i * CPB * C
        pltpu.async_copy(col_hbm.at[pl.ds(eb, CPB * C)], col_b.at[t],
                         cv_sems[t])
        pltpu.async_copy(val_hbm.at[pl.ds(eb, CPB * C)], val_b.at[t],
                         cv_sems[t])

    def cv_wait(i, t):
        eb = edge0 + i * CPB * C
        pltpu.make_async_copy(col_hbm.at[pl.ds(eb, CPB * C)], col_b.at[t],
                              cv_sems[t]).wait()
        pltpu.make_async_copy(val_hbm.at[pl.ds(eb, CPB * C)], val_b.at[t],
                              cv_sems[t]).wait()

    def row_issue(i, t):
        eb = edge0 + i * CPB * C
        for j in range(CPB):
            pltpu.async_copy(row_hbm.at[pl.ds(eb + j * C, C)],
                             row_b.at[t, j], r_sems[t])

    def row_wait(i, t):
        eb = edge0 + i * CPB * C
        for j in range(CPB):
            pltpu.make_async_copy(row_hbm.at[pl.ds(eb + j * C, C)],
                                  row_b.at[t, j], r_sems[t]).wait()

    def g_issue(t):
        for j in range(CPB):
            pltpu.async_copy(x_hbm.at[col_b.at[t, pl.ds(j * C, C)]],
                             gbuf.at[t, j], g_sems[t])

    def g_wait(t):
        for j in range(CPB):
            pltpu.make_async_copy(x_hbm.at[col_b.at[t, pl.ds(j * C, C)]],
                                  gbuf.at[t, j], g_sems[t]).wait()

    def s_issue(t):
        for j in range(CPB):
            pltpu.async_copy(gbuf.at[t, j], acc_sh.at[row_b.at[t, j]],
                             s_sems[t], add=True)

    def s_wait(t):
        for j in range(CPB):
            pltpu.make_async_copy(gbuf.at[t, j], acc_sh.at[row_b.at[t, j]],
                                  s_sems[t]).wait()

    def scale(t):
        for j in range(CPB):
            for g in range(C // 16):
                vv = val_b[t, pl.ds(j * C + g * 16, 16)]
                for e in range(16):
                    i = g * 16 + e
                    bv = jnp.broadcast_to(vv[e], (16,))
                    gbuf[t, j, i, :] = gbuf[t, j, i, :] * bv

    cv_issue(0, 0)
    cv_issue(1, 1)
    row_issue(0, 0)

    @pl.loop(0, ZROWS)
    def _z(i):
        zbuf[i, :] = jnp.zeros((D,), jnp.float32)

    zbase = s * ROWS_PER_SUB
    nfull = ROWS_PER_SUB // ZROWS
    ztail = ROWS_PER_SUB - nfull * ZROWS
    for m in range(nfull):
        pltpu.async_copy(zbuf, acc_sh.at[pl.ds(zbase + m * ZROWS, ZROWS)],
                         zsem)
    if ztail:
        pltpu.async_copy(zbuf.at[pl.ds(0, ztail)],
                         acc_sh.at[pl.ds(zbase + nfull * ZROWS, ztail)], zsem)
    cv_wait(0, 0)
    g_issue(0)
    for m in range(nfull):
        pltpu.make_async_copy(zbuf,
                              acc_sh.at[pl.ds(zbase + m * ZROWS, ZROWS)],
                              zsem).wait()
    if ztail:
        pltpu.make_async_copy(
            zbuf.at[pl.ds(0, ztail)],
            acc_sh.at[pl.ds(zbase + nfull * ZROWS, ztail)], zsem).wait()
    plsc.subcore_barrier()

    @pl.loop(0, HALF)
    def _pair(k):
        for t in (0, 1):
            i = 2 * k + t
            q = 1 - t
            if t == 0:
                @pl.when(k >= 1)
                def _():
                    s_wait(q)
                row_issue(i + 1, q)
                cv_wait(i + 1, q)
                g_issue(q)
            else:
                s_wait(q)

                @pl.when(k < HALF - 1)
                def _():
                    row_issue(i + 1, q)
                    cv_wait(i + 1, q)
                    g_issue(q)
            g_wait(t)
            row_wait(i, t)
            scale(t)
            s_issue(t)

            @pl.when(k < HALF - 1)
            def _():
                cv_issue(i + 2, t)

    s_wait(1)

    plsc.subcore_barrier()
    pltpu.sync_copy(acc_sh.at[pl.ds(s * ROWS_PER_SUB, ROWS_PER_SUB)],
                    out_hbm.at[pl.ds(c * NP + s * ROWS_PER_SUB, ROWS_PER_SUB)])


_RPW = NP // NW
_RB = 184


@functools.partial(
    pl.kernel,
    out_type=jax.ShapeDtypeStruct((NP, D), jnp.float32),
    mesh=_mesh,
    compiler_params=_params,
    scratch_types=[
        pltpu.VMEM((_RB, D), jnp.float32),
        pltpu.VMEM((_RB, D), jnp.float32),
    ],
)
def _combine(p_hbm, out_hbm, a_b, b_b):
    c = lax.axis_index("c")
    s = lax.axis_index("s")
    wid = s * NC + c
    base = wid * _RPW

    @pl.loop(0, _RPW // _RB)
    def _blk(b):
        r0 = base + b * _RB
        pltpu.sync_copy(p_hbm.at[pl.ds(r0, _RB)], a_b)
        pltpu.sync_copy(p_hbm.at[pl.ds(NP + r0, _RB)], b_b)

        @pl.loop(0, _RB)
        def _row(i):
            a_b[i, :] = a_b[i, :] + b_b[i, :]

        pltpu.sync_copy(a_b, out_hbm.at[pl.ds(r0, _RB)])


@functools.partial(
    pl.kernel,
    out_type=jax.ShapeDtypeStruct((NP, D), jnp.float32),
    mesh=_mesh,
    compiler_params=_params,
    scratch_types=[
        pltpu.VMEM((_RB, D), jnp.float32),
        pltpu.VMEM((_RB, D), jnp.float32),
        pltpu.VMEM((_RB, D), jnp.float32),
        pltpu.VMEM((_RB, D), jnp.float32),
    ],
)
def _final_mean(x0_hbm, x1_hbm, q_hbm, out_hbm, a_b, b_b, c_b, d_b):
    c = lax.axis_index("c")
    s = lax.axis_index("s")
    wid = s * NC + c
    base = wid * _RPW
    third = jnp.float32(1.0 / 3.0)

    @pl.loop(0, _RPW // _RB)
    def _blk(b):
        r0 = base + b * _RB
        pltpu.sync_copy(x0_hbm.at[pl.ds(r0, _RB)], a_b)
        pltpu.sync_copy(x1_hbm.at[pl.ds(r0, _RB)], b_b)
        pltpu.sync_copy(q_hbm.at[pl.ds(r0, _RB)], c_b)
        pltpu.sync_copy(q_hbm.at[pl.ds(NP + r0, _RB)], d_b)

        @pl.loop(0, _RB)
        def _row(i):
            acc = (a_b[i, :] + b_b[i, :]) + (c_b[i, :] + d_b[i, :])
            a_b[i, :] = acc * third

        pltpu.sync_copy(a_b, out_hbm.at[pl.ds(r0, _RB)])


def kernel(A_hat_indices, A_hat_values, user_emb, item_emb):
    x0 = jnp.concatenate(
        [user_emb, item_emb, jnp.zeros((NP - N, D), jnp.float32)], axis=0)

    row = A_hat_indices[0].astype(jnp.int32)
    col = A_hat_indices[1].astype(jnp.int32)
    val = A_hat_values.astype(jnp.float32)

    pad = E_PAD - E
    pad_idx = (jnp.arange(pad, dtype=jnp.int32) * 97) % N
    row_p = jnp.concatenate([row, pad_idx])
    col_p = jnp.concatenate([col, pad_idx])
    val_p = jnp.concatenate([val, jnp.zeros((pad,), jnp.float32)])

    p = _propagate(x0, col_p, row_p, val_p)
    x1 = _combine(p)
    q = _propagate(x1, col_p, row_p, val_p)
    out = _final_mean(x0, x1, q)

    return (out[:NUM_USERS], out[NUM_USERS:N])


# speedup vs baseline: 1.2474x; 1.0705x over previous
import functools

import jax
import jax.numpy as jnp
from jax import lax
from jax.experimental import pallas as pl
from jax.experimental.pallas import tpu as pltpu
from jax.experimental.pallas import tpu_sc as plsc

NUM_USERS = 25000
NUM_ITEMS = 75000
N = NUM_USERS + NUM_ITEMS
NP = 102400
E = 3200000
D = 16

NC = 2
NS = 16
NW = NC * NS

C = 128
CPB = 4
CPW = 784
E_PAD = NW * CPW * C
BPW = CPW // CPB
HALF = BPW // 2

ROWS_PER_SUB = NP // NS

_mesh = plsc.VectorSubcoreMesh(core_axis_name="c", subcore_axis_name="s")
_params = pltpu.CompilerParams(use_tc_tiling_on_sc=False)


ZROWS = 256


@functools.partial(
    pl.kernel,
    out_type=jax.ShapeDtypeStruct((2 * NP, D), jnp.float32),
    mesh=_mesh,
    compiler_params=_params,
    scratch_types=[
        pltpu.VMEM_SHARED((NP, D), jnp.float32),
        pltpu.VMEM((2, CPB * C), jnp.int32),
        pltpu.VMEM((2, CPB, C), jnp.int32),
        pltpu.VMEM((2, CPB * C), jnp.float32),
        pltpu.VMEM((2, CPB, C, D), jnp.float32),
        pltpu.VMEM((ZROWS, D), jnp.float32),
        pltpu.SemaphoreType.DMA,
        pltpu.SemaphoreType.DMA,
        pltpu.SemaphoreType.DMA,
        pltpu.SemaphoreType.DMA,
        pltpu.SemaphoreType.DMA,
        pltpu.SemaphoreType.DMA,
        pltpu.SemaphoreType.DMA,
        pltpu.SemaphoreType.DMA,
        pltpu.SemaphoreType.DMA,
    ],
)
def _propagate(x_hbm, col_hbm, row_hbm, val_hbm, out_hbm,
               acc_sh, col_b, row_b, val_b, gbuf, zbuf,
               zsem, cv0, cv1, r0, r1, g0, g1, s0, s1):
    cv_sems, r_sems = (cv0, cv1), (r0, r1)
    g_sems, s_sems = (g0, g1), (s0, s1)
    c = lax.axis_index("c")
    s = lax.axis_index("s")
    wid = s * NC + c

    edge0 = wid * CPW * C

    def cv_issue(i, t):
        eb = edge0 + i * CPB * C
        pltpu.async_copy(col_hbm.at[pl.ds(eb, CPB * C)], col_b.at[t],
                         cv_sems[t])
        pltpu.async_copy(val_hbm.at[pl.ds(eb, CPB * C)], val_b.at[t],
                         cv_sems[t])

    def cv_wait(i, t):
        eb = edge0 + i * CPB * C
        pltpu.make_async_copy(col_hbm.at[pl.ds(eb, CPB * C)], col_b.at[t],
                              cv_sems[t]).wait()
        pltpu.make_async_copy(val_hbm.at[pl.ds(eb, CPB * C)], val_b.at[t],
                              cv_sems[t]).wait()

    def row_issue(i, t):
        eb = edge0 + i * CPB * C
        for j in range(CPB):
            pltpu.async_copy(row_hbm.at[pl.ds(eb + j * C, C)],
                             row_b.at[t, j], r_sems[t])

    def row_wait(i, t):
        eb = edge0 + i * CPB * C
        for j in range(CPB):
            pltpu.make_async_copy(row_hbm.at[pl.ds(eb + j * C, C)],
                                  row_b.at[t, j], r_sems[t]).wait()

    def g_issue(t):
        for j in range(CPB):
            pltpu.async_copy(x_hbm.at[col_b.at[t, pl.ds(j * C, C)]],
                             gbuf.at[t, j], g_sems[t])

    def g_wait(t):
        for j in range(CPB):
            pltpu.make_async_copy(x_hbm.at[col_b.at[t, pl.ds(j * C, C)]],
                                  gbuf.at[t, j], g_sems[t]).wait()

    def s_issue(t):
        for j in range(CPB):
            pltpu.async_copy(gbuf.at[t, j], acc_sh.at[row_b.at[t, j]],
                             s_sems[t], add=True)

    def s_wait(t):
        for j in range(CPB):
            pltpu.make_async_copy(gbuf.at[t, j], acc_sh.at[row_b.at[t, j]],
                                  s_sems[t]).wait()

    def scale(t):
        for j in range(CPB):
            for g in range(C // 16):
                vv = val_b[t, pl.ds(j * C + g * 16, 16)]
                for e in range(16):
                    i = g * 16 + e
                    bv = jnp.broadcast_to(vv[e], (16,))
                    gbuf[t, j, i, :] = gbuf[t, j, i, :] * bv

    cv_issue(0, 0)
    cv_issue(1, 1)
    row_issue(0, 0)

    @pl.loop(0, ZROWS)
    def _z(i):
        zbuf[i, :] = jnp.zeros((D,), jnp.float32)

    zbase = s * ROWS_PER_SUB
    nfull = ROWS_PER_SUB // ZROWS
    ztail = ROWS_PER_SUB - nfull * ZROWS
    for m in range(nfull):
        pltpu.async_copy(zbuf, acc_sh.at[pl.ds(zbase + m * ZROWS, ZROWS)],
                         zsem)
    if ztail:
        pltpu.async_copy(zbuf.at[pl.ds(0, ztail)],
                         acc_sh.at[pl.ds(zbase + nfull * ZROWS, ztail)], zsem)
    cv_wait(0, 0)
    g_issue(0)
    for m in range(nfull):
        pltpu.make_async_copy(zbuf,
                              acc_sh.at[pl.ds(zbase + m * ZROWS, ZROWS)],
                              zsem).wait()
    if ztail:
        pltpu.make_async_copy(
            zbuf.at[pl.ds(0, ztail)],
            acc_sh.at[pl.ds(zbase + nfull * ZROWS, ztail)], zsem).wait()
    plsc.subcore_barrier()

    @pl.loop(0, HALF)
    def _pair(k):
        for t in (0, 1):
            i = 2 * k + t
            q = 1 - t
            if t == 0:
                @pl.when(k >= 1)
                def _():
                    s_wait(q)
                row_issue(i + 1, q)
                cv_wait(i + 1, q)
                g_issue(q)
            else:
                s_wait(q)

                @pl.when(k < HALF - 1)
                def _():
                    row_issue(i + 1, q)
                    cv_wait(i + 1, q)
                    g_issue(q)
            g_wait(t)
            row_wait(i, t)
            scale(t)
            s_issue(t)

            @pl.when(k < HALF - 1)
            def _():
                cv_issue(i + 2, t)

    s_wait(1)

    plsc.subcore_barrier()
    pltpu.sync_copy(acc_sh.at[pl.ds(s * ROWS_PER_SUB, ROWS_PER_SUB)],
                    out_hbm.at[pl.ds(c * NP + s * ROWS_PER_SUB, ROWS_PER_SUB)])


_RPW = NP // NW
_RB = 800


@functools.partial(
    pl.kernel,
    out_type=jax.ShapeDtypeStruct((NP, D), jnp.float32),
    mesh=_mesh,
    compiler_params=_params,
    scratch_types=[
        pltpu.VMEM((_RB, D), jnp.float32),
        pltpu.VMEM((_RB, D), jnp.float32),
        pltpu.SemaphoreType.DMA,
    ],
)
def _combine(p_hbm, out_hbm, a_b, b_b, ldsem):
    c = lax.axis_index("c")
    s = lax.axis_index("s")
    wid = s * NC + c
    base = wid * _RPW

    @pl.loop(0, _RPW // _RB)
    def _blk(b):
        r0 = base + b * _RB
        h1 = pltpu.async_copy(p_hbm.at[pl.ds(r0, _RB)], a_b, ldsem)
        h2 = pltpu.async_copy(p_hbm.at[pl.ds(NP + r0, _RB)], b_b, ldsem)
        h1.wait()
        h2.wait()

        @pl.loop(0, _RB)
        def _row(i):
            a_b[i, :] = a_b[i, :] + b_b[i, :]

        pltpu.sync_copy(a_b, out_hbm.at[pl.ds(r0, _RB)])


@functools.partial(
    pl.kernel,
    out_type=jax.ShapeDtypeStruct((NP, D), jnp.float32),
    mesh=_mesh,
    compiler_params=_params,
    scratch_types=[
        pltpu.VMEM((_RB, D), jnp.float32),
        pltpu.VMEM((_RB, D), jnp.float32),
        pltpu.VMEM((_RB, D), jnp.float32),
        pltpu.VMEM((_RB, D), jnp.float32),
        pltpu.SemaphoreType.DMA,
    ],
)
def _final_mean(x0_hbm, x1_hbm, q_hbm, out_hbm, a_b, b_b, c_b, d_b, ldsem):
    c = lax.axis_index("c")
    s = lax.axis_index("s")
    wid = s * NC + c
    base = wid * _RPW
    third = jnp.float32(1.0 / 3.0)

    @pl.loop(0, _RPW // _RB)
    def _blk(b):
        r0 = base + b * _RB
        h1 = pltpu.async_copy(x0_hbm.at[pl.ds(r0, _RB)], a_b, ldsem)
        h2 = pltpu.async_copy(x1_hbm.at[pl.ds(r0, _RB)], b_b, ldsem)
        h3 = pltpu.async_copy(q_hbm.at[pl.ds(r0, _RB)], c_b, ldsem)
        h4 = pltpu.async_copy(q_hbm.at[pl.ds(NP + r0, _RB)], d_b, ldsem)
        h1.wait()
        h2.wait()
        h3.wait()
        h4.wait()

        @pl.loop(0, _RB)
        def _row(i):
            acc = (a_b[i, :] + b_b[i, :]) + (c_b[i, :] + d_b[i, :])
            a_b[i, :] = acc * third

        pltpu.sync_copy(a_b, out_hbm.at[pl.ds(r0, _RB)])


def kernel(A_hat_indices, A_hat_values, user_emb, item_emb):
    x0 = jnp.concatenate(
        [user_emb, item_emb, jnp.zeros((NP - N, D), jnp.float32)], axis=0)

    row = A_hat_indices[0].astype(jnp.int32)
    col = A_hat_indices[1].astype(jnp.int32)
    val = A_hat_values.astype(jnp.float32)

    pad = E_PAD - E
    pad_idx = (jnp.arange(pad, dtype=jnp.int32) * 97) % N
    row_p = jnp.concatenate([row, pad_idx])
    col_p = jnp.concatenate([col, pad_idx])
    val_p = jnp.concatenate([val, jnp.zeros((pad,), jnp.float32)])

    p = _propagate(x0, col_p, row_p, val_p)
    x1 = _combine(p)
    q = _propagate(x1, col_p, row_p, val_p)
    out = _final_mean(x0, x1, q)

    return (out[:NUM_USERS], out[NUM_USERS:N])
